# baseline (speedup 1.0000x reference)
import functools

import jax
import jax.numpy as jnp
import numpy as np
from jax import lax
from jax.experimental import pallas as pl
from jax.experimental.pallas import tpu as pltpu
from jax.experimental.pallas import tpu_sc as plsc

S = 2048
D_MODEL = 1024
NUM_HEADS = 16
NUM_KV_HEADS = 4
HEAD_DIM = 64
N_REP = NUM_HEADS // NUM_KV_HEADS
KV_D = NUM_KV_HEADS * HEAD_DIM
_PID = (0, 1, 2, 3, 50, 100)
_LAYER_IDX = 8
_KK = int(192 - _LAYER_IDX / 31 * (192 - 64))

_ZROWS = 512
_NV = S // 16


def _bf16(a):
    return a.astype(jnp.bfloat16)


def _mega_kernel(
    x_hbm, wq_hbm, wkv_hbm,
    pruned_ref, out_ref, kv_hbm, attn_hbm, imp_hbm,
    xbuf, wqbuf, wkvbuf, kvbuf, zbuf, attnbuf, impbuf, sems,
):
    in_x = pltpu.make_async_copy(x_hbm, xbuf, sems.at[120])
    in_q = pltpu.make_async_copy(wq_hbm, wqbuf, sems.at[121])
    in_kv = pltpu.make_async_copy(wkv_hbm, wkvbuf, sems.at[122])
    for c in (in_x, in_kv, in_q):
        c.start()

    zbuf[...] = jnp.zeros_like(zbuf)
    copies = []
    n = 0
    nblk = (S - 8) // _ZROWS
    rem = (S - 8) - nblk * _ZROWS
    for h in range(NUM_HEADS):
        for j in range(nblk):
            c = pltpu.make_async_copy(
                zbuf,
                pruned_ref.at[h, j * _ZROWS : (j + 1) * _ZROWS, :],
                sems.at[n],
            )
            c.start()
            copies.append(c)
            n += 1
        c = pltpu.make_async_copy(
            zbuf.at[0:rem, :],
            pruned_ref.at[h, nblk * _ZROWS : S - 8, :],
            sems.at[n],
        )
        c.start()
        copies.append(c)
        n += 1
    for j in range(nblk):
        c = pltpu.make_async_copy(
            zbuf.at[:, 0:D_MODEL],
            out_ref.at[j * _ZROWS : (j + 1) * _ZROWS, :],
            sems.at[n],
        )
        c.start()
        copies.append(c)
        n += 1
    c = pltpu.make_async_copy(
        zbuf.at[0:rem, 0:D_MODEL],
        out_ref.at[nblk * _ZROWS : S - 8, :],
        sems.at[n],
    )
    c.start()
    copies.append(c)
    n += 1

    in_x.wait()
    in_kv.wait()
    half = S // 2
    for i in range(2):
        kvbuf[i * half : (i + 1) * half, :] = jnp.dot(
            _bf16(xbuf[i * half : (i + 1) * half, :]),
            _bf16(wkvbuf[...]),
            preferred_element_type=jnp.float32,
        )
        c = pltpu.make_async_copy(
            kvbuf.at[i * half : (i + 1) * half, :],
            kv_hbm.at[i * half : (i + 1) * half, :],
            sems.at[n],
        )
        c.start()
        copies.append(c)
        n += 1
    k = kvbuf[:, :KV_D]

    in_q.wait()
    q = jnp.dot(
        _bf16(xbuf[S - 1 : S, :]),
        _bf16(wqbuf[...]),
        preferred_element_type=jnp.float32,
    )
    rows = []
    for h in range(NUM_HEADS):
        qh = _bf16(q[:, h * HEAD_DIM : (h + 1) * HEAD_DIM])
        g = h // N_REP
        kg = _bf16(k[:, g * HEAD_DIM : (g + 1) * HEAD_DIM])
        rows.append(
            jax.lax.dot_general(
                qh, kg, (((1,), (1,)), ((), ())),
                preferred_element_type=jnp.float32,
            )
        )
    scores = jnp.concatenate(rows, axis=0) * (
        1.0 / np.sqrt(HEAD_DIM)
    )
    m = jnp.max(scores, axis=1, keepdims=True)
    e = jnp.exp(scores - m)
    attn = e / jnp.sum(e, axis=1, keepdims=True)
    attnbuf[...] = attn

    imp = jnp.mean(attn, axis=0, keepdims=True)
    lane = jax.lax.broadcasted_iota(jnp.int32, (1, S), 1)
    is_pid = functools.reduce(jnp.logical_or, [lane == p for p in _PID])
    impbuf[...] = jnp.where(is_pid, imp * 2.5, imp)

    c = pltpu.make_async_copy(attnbuf, attn_hbm, sems.at[n])
    c.start()
    copies.append(c)
    n += 1
    c = pltpu.make_async_copy(impbuf, imp_hbm, sems.at[n])
    c.start()
    copies.append(c)
    n += 1

    for c in copies:
        c.wait()


def _sc_topk_kernel(imp_hbm, sel_hbm, impv, bitsv, selv):
    cid = lax.axis_index("c")
    sid = lax.axis_index("s")

    @pl.when(jnp.logical_and(cid == 0, sid == 0))
    def _():
        pltpu.sync_copy(imp_hbm, impv)

        def tobits(i, carry):
            v = impv[pl.ds(i * 16, 16)]
            bitsv[pl.ds(i * 16, 16)] = plsc.bitcast(v, jnp.int32)
            return carry

        lax.fori_loop(0, _NV, tobits, jnp.int32(0))

        def count_ge(cand):
            def inner(i, cnt):
                bv = bitsv[pl.ds(i * 16, 16)]
                ones = jnp.where(bv >= cand, jnp.int32(1), jnp.int32(0))
                return cnt + jnp.sum(ones)

            return lax.fori_loop(0, _NV, inner, jnp.int32(0))

        def outer(b, t):
            cand = t | lax.shift_left(jnp.int32(1), 30 - b)
            return jnp.where(count_ge(cand) >= _KK, cand, t)

        t = lax.fori_loop(0, 31, outer, jnp.int32(0))

        ngt = count_ge(t + 1)
        need = _KK - ngt

        def selpass(i, run):
            bv = bitsv[pl.ds(i * 16, 16)]
            eqv = bv == t
            ones = jnp.where(eqv, jnp.int32(1), jnp.int32(0))
            cs = lax.cumsum(ones) + run
            keep = jnp.logical_or(
                bv > t, jnp.logical_and(eqv, cs <= need)
            )
            selv[pl.ds(i * 16, 16)] = jnp.where(keep, 1.0, 0.0)
            return run + jnp.sum(ones)

        lax.fori_loop(0, _NV, selpass, jnp.int32(0))
        pltpu.sync_copy(selv, sel_hbm)


def _sc_topk(imp):
    mesh = plsc.VectorSubcoreMesh(core_axis_name="c", subcore_axis_name="s")
    kfn = functools.partial(
        pl.kernel,
        mesh=mesh,
        out_type=jax.ShapeDtypeStruct((S,), jnp.float32),
        scratch_types=[
            pltpu.VMEM((S,), jnp.float32),
            pltpu.VMEM((S,), jnp.int32),
            pltpu.VMEM((S,), jnp.float32),
        ],
        compiler_params=pltpu.CompilerParams(needs_layout_passes=False),
    )(_sc_topk_kernel)
    return kfn(imp)


def _tail_kernel(
    attn_ref, sel_ref, kv_hbm, wo_hbm,
    pruned_in, out_in,
    pruned_ref, out_ref,
    vbuf, wobuf, prow_buf, olast_buf, sems,
):
    del pruned_in, out_in
    in_v = pltpu.make_async_copy(
        kv_hbm.at[:, KV_D : 2 * KV_D], vbuf, sems.at[30]
    )
    in_o = pltpu.make_async_copy(wo_hbm, wobuf, sems.at[31])
    in_v.start()
    in_o.start()

    prow = attn_ref[...] * sel_ref[...]
    prow_buf[...] = jnp.zeros_like(prow_buf)
    for h in range(NUM_HEADS):
        prow_buf[8 * h + 7 : 8 * h + 8, :] = prow[h : h + 1, :]

    copies = []
    n = 0
    for h in range(NUM_HEADS):
        c = pltpu.make_async_copy(
            prow_buf.at[8 * h : 8 * (h + 1), :],
            pruned_ref.at[h, S - 8 : S, :],
            sems.at[n],
        )
        c.start()
        copies.append(c)
        n += 1

    in_v.wait()
    ctx = jnp.dot(
        _bf16(prow), _bf16(vbuf[...]), preferred_element_type=jnp.float32
    )
    hh = jax.lax.broadcasted_iota(jnp.int32, (NUM_HEADS, KV_D), 0)
    gg = jax.lax.broadcasted_iota(jnp.int32, (NUM_HEADS, KV_D), 1) // HEAD_DIM
    ctx = jnp.where(hh // N_REP == gg, ctx, 0.0)
    ctx16 = (
        ctx[:, 0:64] + ctx[:, 64:128] + ctx[:, 128:192] + ctx[:, 192:256]
    )

    in_o.wait()
    olast = jnp.zeros((1, D_MODEL), jnp.float32)
    for h in range(NUM_HEADS):
        olast = olast + jnp.dot(
            _bf16(ctx16[h : h + 1, :]),
            _bf16(wobuf[h * HEAD_DIM : (h + 1) * HEAD_DIM, :]),
            preferred_element_type=jnp.float32,
        )
    olast_buf[...] = jnp.zeros_like(olast_buf)
    olast_buf[7:8, :] = olast

    c = pltpu.make_async_copy(olast_buf, out_ref.at[S - 8 : S, :], sems.at[n])
    c.start()
    copies.append(c)
    n += 1

    for c in copies:
        c.wait()


def kernel(hidden_states, Wq, Wk, Wv, Wo):
    x = hidden_states[0]
    Wkv = jnp.concatenate([Wk, Wv], axis=1)

    hbm = pl.BlockSpec(memory_space=pltpu.MemorySpace.HBM)
    vmem = pl.BlockSpec(memory_space=pltpu.MemorySpace.VMEM)
    pruned0, out0, kv, attn, imp = pl.pallas_call(
        _mega_kernel,
        in_specs=[hbm, hbm, hbm],
        out_specs=(hbm, hbm, hbm, hbm, hbm),
        out_shape=(
            jax.ShapeDtypeStruct((NUM_HEADS, S, S), jnp.float32),
            jax.ShapeDtypeStruct((S, D_MODEL), jnp.float32),
            jax.ShapeDtypeStruct((S, 2 * KV_D), jnp.float32),
            jax.ShapeDtypeStruct((NUM_HEADS, S), jnp.float32),
            jax.ShapeDtypeStruct((1, S), jnp.float32),
        ),
        scratch_shapes=[
            pltpu.VMEM((S, D_MODEL), jnp.float32),
            pltpu.VMEM((D_MODEL, D_MODEL), jnp.float32),
            pltpu.VMEM((D_MODEL, 2 * KV_D), jnp.float32),
            pltpu.VMEM((S, 2 * KV_D), jnp.float32),
            pltpu.VMEM((_ZROWS, S), jnp.float32),
            pltpu.VMEM((NUM_HEADS, S), jnp.float32),
            pltpu.VMEM((1, S), jnp.float32),
            pltpu.SemaphoreType.DMA((128,)),
        ],
    )(x, Wq, Wkv)

    sel = _sc_topk(imp.reshape(S))

    pruned, out = pl.pallas_call(
        _tail_kernel,
        in_specs=[vmem, vmem, hbm, hbm, hbm, hbm],
        out_specs=(hbm, hbm),
        out_shape=(
            jax.ShapeDtypeStruct((NUM_HEADS, S, S), jnp.float32),
            jax.ShapeDtypeStruct((S, D_MODEL), jnp.float32),
        ),
        input_output_aliases={4: 0, 5: 1},
        scratch_shapes=[
            pltpu.VMEM((S, KV_D), jnp.float32),
            pltpu.VMEM((D_MODEL, D_MODEL), jnp.float32),
            pltpu.VMEM((8 * NUM_HEADS, S), jnp.float32),
            pltpu.VMEM((8, D_MODEL), jnp.float32),
            pltpu.SemaphoreType.DMA((64,)),
        ],
    )(attn, sel.reshape(1, S), kv, Wo, pruned0, out0)

    k_flat = kv[:, :KV_D]
    v_flat = kv[:, KV_D:]
    k_kv = k_flat.reshape(1, S, NUM_KV_HEADS, HEAD_DIM).transpose(0, 2, 1, 3)
    v_kv = v_flat.reshape(1, S, NUM_KV_HEADS, HEAD_DIM).transpose(0, 2, 1, 3)
    return out[None], pruned[None], k_kv, v_kv


# speedup vs baseline: 1.0763x; 1.0763x over previous
import functools

import jax
import jax.numpy as jnp
import numpy as np
from jax import lax
from jax.experimental import pallas as pl
from jax.experimental.pallas import tpu as pltpu
from jax.experimental.pallas import tpu_sc as plsc

S = 2048
D_MODEL = 1024
NUM_HEADS = 16
NUM_KV_HEADS = 4
HEAD_DIM = 64
N_REP = NUM_HEADS // NUM_KV_HEADS
KV_D = NUM_KV_HEADS * HEAD_DIM
_PID = (0, 1, 2, 3, 50, 100)
_LAYER_IDX = 8
_KK = int(192 - _LAYER_IDX / 31 * (192 - 64))

_ZROWS = 512
_NV = S // 16


def _bf16(a):
    return a.astype(jnp.bfloat16)


def _mega_kernel(
    x_hbm, wq_hbm, wkv_hbm,
    pruned_ref, out_ref, kv_hbm, attn_hbm, imp_hbm,
    xbuf, wqbuf, wkvbuf, kvbuf, zbuf, attnbuf, impbuf, sems,
):
    in_x = pltpu.make_async_copy(x_hbm, xbuf, sems.at[120])
    in_q = pltpu.make_async_copy(wq_hbm, wqbuf, sems.at[121])
    in_kv = pltpu.make_async_copy(wkv_hbm, wkvbuf, sems.at[122])
    for c in (in_x, in_kv, in_q):
        c.start()

    zbuf[...] = jnp.zeros_like(zbuf)
    copies = []
    n = 0
    nblk = (S - 8) // _ZROWS
    rem = (S - 8) - nblk * _ZROWS
    for h in range(NUM_HEADS):
        for j in range(nblk):
            c = pltpu.make_async_copy(
                zbuf,
                pruned_ref.at[h, j * _ZROWS : (j + 1) * _ZROWS, :],
                sems.at[n],
            )
            c.start()
            copies.append(c)
            n += 1
        c = pltpu.make_async_copy(
            zbuf.at[0:rem, :],
            pruned_ref.at[h, nblk * _ZROWS : S - 8, :],
            sems.at[n],
        )
        c.start()
        copies.append(c)
        n += 1
    for j in range(nblk):
        c = pltpu.make_async_copy(
            zbuf.at[:, 0:D_MODEL],
            out_ref.at[j * _ZROWS : (j + 1) * _ZROWS, :],
            sems.at[n],
        )
        c.start()
        copies.append(c)
        n += 1
    c = pltpu.make_async_copy(
        zbuf.at[0:rem, 0:D_MODEL],
        out_ref.at[nblk * _ZROWS : S - 8, :],
        sems.at[n],
    )
    c.start()
    copies.append(c)
    n += 1

    in_x.wait()
    in_kv.wait()
    half = S // 2
    for i in range(2):
        kvbuf[i * half : (i + 1) * half, :] = jnp.dot(
            _bf16(xbuf[i * half : (i + 1) * half, :]),
            _bf16(wkvbuf[...]),
            preferred_element_type=jnp.float32,
        )
        c = pltpu.make_async_copy(
            kvbuf.at[i * half : (i + 1) * half, :],
            kv_hbm.at[i * half : (i + 1) * half, :],
            sems.at[n],
        )
        c.start()
        copies.append(c)
        n += 1
    k = kvbuf[:, :KV_D]

    in_q.wait()
    q = jnp.dot(
        _bf16(xbuf[S - 1 : S, :]),
        _bf16(wqbuf[...]),
        preferred_element_type=jnp.float32,
    )
    rows = []
    for h in range(NUM_HEADS):
        qh = _bf16(q[:, h * HEAD_DIM : (h + 1) * HEAD_DIM])
        g = h // N_REP
        kg = _bf16(k[:, g * HEAD_DIM : (g + 1) * HEAD_DIM])
        rows.append(
            jax.lax.dot_general(
                qh, kg, (((1,), (1,)), ((), ())),
                preferred_element_type=jnp.float32,
            )
        )
    scores = jnp.concatenate(rows, axis=0) * (
        1.0 / np.sqrt(HEAD_DIM)
    )
    m = jnp.max(scores, axis=1, keepdims=True)
    e = jnp.exp(scores - m)
    attn = e / jnp.sum(e, axis=1, keepdims=True)
    attnbuf[...] = attn

    imp = jnp.mean(attn, axis=0, keepdims=True)
    lane = jax.lax.broadcasted_iota(jnp.int32, (1, S), 1)
    is_pid = functools.reduce(jnp.logical_or, [lane == p for p in _PID])
    impbuf[...] = jnp.where(is_pid, imp * 2.5, imp)

    c = pltpu.make_async_copy(attnbuf, attn_hbm, sems.at[n])
    c.start()
    copies.append(c)
    n += 1
    c = pltpu.make_async_copy(impbuf, imp_hbm, sems.at[n])
    c.start()
    copies.append(c)
    n += 1

    for c in copies:
        c.wait()


def _sc_topk_kernel(imp_hbm, sel_hbm, impv, bitsv, selv):
    cid = lax.axis_index("c")
    sid = lax.axis_index("s")

    @pl.when(jnp.logical_and(cid == 0, sid == 0))
    def _():
        pltpu.sync_copy(imp_hbm, impv)

        def tobits(i, carry):
            for u in range(4):
                v = impv[pl.ds((4 * i + u) * 16, 16)]
                bitsv[pl.ds((4 * i + u) * 16, 16)] = plsc.bitcast(
                    v, jnp.int32
                )
            return carry

        lax.fori_loop(0, _NV // 4, tobits, jnp.int32(0))

        def count_ge(cand):
            def inner(i, cnt):
                acc = jnp.zeros((16,), jnp.int32)
                for u in range(4):
                    bv = bitsv[pl.ds((4 * i + u) * 16, 16)]
                    acc = acc + jnp.where(
                        bv >= cand, jnp.int32(1), jnp.int32(0)
                    )
                return cnt + jnp.sum(acc)

            return lax.fori_loop(0, _NV // 4, inner, jnp.int32(0))

        def outer(b, t):
            cand = t | lax.shift_left(jnp.int32(1), 30 - b)
            return jnp.where(count_ge(cand) >= _KK, cand, t)

        t = lax.fori_loop(0, 31, outer, jnp.int32(0))

        ngt = count_ge(t + 1)
        need = _KK - ngt

        def selpass(i, run):
            bv = bitsv[pl.ds(i * 16, 16)]
            eqv = bv == t
            ones = jnp.where(eqv, jnp.int32(1), jnp.int32(0))
            cs = lax.cumsum(ones) + run
            keep = jnp.logical_or(
                bv > t, jnp.logical_and(eqv, cs <= need)
            )
            selv[pl.ds(i * 16, 16)] = jnp.where(keep, 1.0, 0.0)
            return run + jnp.sum(ones)

        lax.fori_loop(0, _NV, selpass, jnp.int32(0))
        pltpu.sync_copy(selv, sel_hbm)


def _sc_topk(imp):
    mesh = plsc.VectorSubcoreMesh(core_axis_name="c", subcore_axis_name="s")
    kfn = functools.partial(
        pl.kernel,
        mesh=mesh,
        out_type=jax.ShapeDtypeStruct((S,), jnp.float32),
        scratch_types=[
            pltpu.VMEM((S,), jnp.float32),
            pltpu.VMEM((S,), jnp.int32),
            pltpu.VMEM((S,), jnp.float32),
        ],
        compiler_params=pltpu.CompilerParams(needs_layout_passes=False),
    )(_sc_topk_kernel)
    return kfn(imp)


def _tail_kernel(
    attn_ref, sel_ref, kv_hbm, wo_hbm,
    pruned_in, out_in,
    pruned_ref, out_ref,
    vbuf, wobuf, prow_buf, olast_buf, sems,
):
    del pruned_in, out_in
    in_v = pltpu.make_async_copy(
        kv_hbm.at[:, KV_D : 2 * KV_D], vbuf, sems.at[30]
    )
    in_o = pltpu.make_async_copy(wo_hbm, wobuf, sems.at[31])
    in_v.start()
    in_o.start()

    prow = attn_ref[...] * sel_ref[...]
    prow_buf[...] = jnp.zeros_like(prow_buf)
    for h in range(NUM_HEADS):
        prow_buf[8 * h + 7 : 8 * h + 8, :] = prow[h : h + 1, :]

    copies = []
    n = 0
    for h in range(NUM_HEADS):
        c = pltpu.make_async_copy(
            prow_buf.at[8 * h : 8 * (h + 1), :],
            pruned_ref.at[h, S - 8 : S, :],
            sems.at[n],
        )
        c.start()
        copies.append(c)
        n += 1

    in_v.wait()
    ctx = jnp.dot(
        _bf16(prow), _bf16(vbuf[...]), preferred_element_type=jnp.float32
    )
    hh = jax.lax.broadcasted_iota(jnp.int32, (NUM_HEADS, KV_D), 0)
    gg = jax.lax.broadcasted_iota(jnp.int32, (NUM_HEADS, KV_D), 1) // HEAD_DIM
    ctx = jnp.where(hh // N_REP == gg, ctx, 0.0)
    ctx16 = (
        ctx[:, 0:64] + ctx[:, 64:128] + ctx[:, 128:192] + ctx[:, 192:256]
    )

    in_o.wait()
    olast = jnp.zeros((1, D_MODEL), jnp.float32)
    for h in range(NUM_HEADS):
        olast = olast + jnp.dot(
            _bf16(ctx16[h : h + 1, :]),
            _bf16(wobuf[h * HEAD_DIM : (h + 1) * HEAD_DIM, :]),
            preferred_element_type=jnp.float32,
        )
    olast_buf[...] = jnp.zeros_like(olast_buf)
    olast_buf[7:8, :] = olast

    c = pltpu.make_async_copy(olast_buf, out_ref.at[S - 8 : S, :], sems.at[n])
    c.start()
    copies.append(c)
    n += 1

    for c in copies:
        c.wait()


def kernel(hidden_states, Wq, Wk, Wv, Wo):
    x = hidden_states[0]
    Wkv = jnp.concatenate([Wk, Wv], axis=1)

    hbm = pl.BlockSpec(memory_space=pltpu.MemorySpace.HBM)
    vmem = pl.BlockSpec(memory_space=pltpu.MemorySpace.VMEM)
    pruned0, out0, kv, attn, imp = pl.pallas_call(
        _mega_kernel,
        in_specs=[hbm, hbm, hbm],
        out_specs=(hbm, hbm, hbm, hbm, hbm),
        out_shape=(
            jax.ShapeDtypeStruct((NUM_HEADS, S, S), jnp.float32),
            jax.ShapeDtypeStruct((S, D_MODEL), jnp.float32),
            jax.ShapeDtypeStruct((S, 2 * KV_D), jnp.float32),
            jax.ShapeDtypeStruct((NUM_HEADS, S), jnp.float32),
            jax.ShapeDtypeStruct((1, S), jnp.float32),
        ),
        scratch_shapes=[
            pltpu.VMEM((S, D_MODEL), jnp.float32),
            pltpu.VMEM((D_MODEL, D_MODEL), jnp.float32),
            pltpu.VMEM((D_MODEL, 2 * KV_D), jnp.float32),
            pltpu.VMEM((S, 2 * KV_D), jnp.float32),
            pltpu.VMEM((_ZROWS, S), jnp.float32),
            pltpu.VMEM((NUM_HEADS, S), jnp.float32),
            pltpu.VMEM((1, S), jnp.float32),
            pltpu.SemaphoreType.DMA((128,)),
        ],
    )(x, Wq, Wkv)

    sel = _sc_topk(imp.reshape(S))

    pruned, out = pl.pallas_call(
        _tail_kernel,
        in_specs=[vmem, vmem, hbm, hbm, hbm, hbm],
        out_specs=(hbm, hbm),
        out_shape=(
            jax.ShapeDtypeStruct((NUM_HEADS, S, S), jnp.float32),
            jax.ShapeDtypeStruct((S, D_MODEL), jnp.float32),
        ),
        input_output_aliases={4: 0, 5: 1},
        scratch_shapes=[
            pltpu.VMEM((S, KV_D), jnp.float32),
            pltpu.VMEM((D_MODEL, D_MODEL), jnp.float32),
            pltpu.VMEM((8 * NUM_HEADS, S), jnp.float32),
            pltpu.VMEM((8, D_MODEL), jnp.float32),
            pltpu.SemaphoreType.DMA((64,)),
        ],
    )(attn, sel.reshape(1, S), kv, Wo, pruned0, out0)

    k_flat = kv[:, :KV_D]
    v_flat = kv[:, KV_D:]
    k_kv = k_flat.reshape(1, S, NUM_KV_HEADS, HEAD_DIM).transpose(0, 2, 1, 3)
    v_kv = v_flat.reshape(1, S, NUM_KV_HEADS, HEAD_DIM).transpose(0, 2, 1, 3)
    return out[None], pruned[None], k_kv, v_kv


# speedup vs baseline: 1.0780x; 1.0016x over previous
import functools

import jax
import jax.numpy as jnp
import numpy as np
from jax import lax
from jax.experimental import pallas as pl
from jax.experimental.pallas import tpu as pltpu
from jax.experimental.pallas import tpu_sc as plsc

S = 2048
D_MODEL = 1024
NUM_HEADS = 16
NUM_KV_HEADS = 4
HEAD_DIM = 64
N_REP = NUM_HEADS // NUM_KV_HEADS
KV_D = NUM_KV_HEADS * HEAD_DIM
_PID = (0, 1, 2, 3, 50, 100)
_LAYER_IDX = 8
_KK = int(192 - _LAYER_IDX / 31 * (192 - 64))

_ZROWS = 512
_NV = S // 16


def _bf16(a):
    return a.astype(jnp.bfloat16)


def _mega_kernel(
    x_hbm, wq_hbm, wkv_hbm,
    pruned_ref, out_ref, kv_hbm, attn_hbm, imp_hbm,
    xbuf, wqbuf, wkvbuf, kvbuf, zbuf, zbuf2, attnbuf, impbuf, sems,
):
    in_x = pltpu.make_async_copy(x_hbm, xbuf, sems.at[120])
    in_q = pltpu.make_async_copy(wq_hbm, wqbuf, sems.at[121])
    in_kv = pltpu.make_async_copy(wkv_hbm, wkvbuf, sems.at[122])
    for c in (in_x, in_kv, in_q):
        c.start()

    zbuf[...] = jnp.zeros_like(zbuf)
    zbuf2[...] = jnp.zeros_like(zbuf2)
    zsrcs = (zbuf, zbuf2)
    copies = []
    n = 0
    nblk = (S - 8) // _ZROWS
    rem = (S - 8) - nblk * _ZROWS
    for h in range(NUM_HEADS):
        for j in range(nblk):
            zsrc = zsrcs[(h * (nblk + 1) + j) % 2]
            c = pltpu.make_async_copy(
                zsrc,
                pruned_ref.at[h, j * _ZROWS : (j + 1) * _ZROWS, :],
                sems.at[n],
            )
            c.start()
            copies.append(c)
            n += 1
        zsrc = zsrcs[(h * (nblk + 1) + nblk) % 2]
        c = pltpu.make_async_copy(
            zsrc.at[0:rem, :],
            pruned_ref.at[h, nblk * _ZROWS : S - 8, :],
            sems.at[n],
        )
        c.start()
        copies.append(c)
        n += 1
    for j in range(nblk):
        c = pltpu.make_async_copy(
            zsrcs[j % 2].at[:, 0:D_MODEL],
            out_ref.at[j * _ZROWS : (j + 1) * _ZROWS, :],
            sems.at[n],
        )
        c.start()
        copies.append(c)
        n += 1
    c = pltpu.make_async_copy(
        zbuf.at[0:rem, 0:D_MODEL],
        out_ref.at[nblk * _ZROWS : S - 8, :],
        sems.at[n],
    )
    c.start()
    copies.append(c)
    n += 1

    in_x.wait()
    in_kv.wait()
    half = S // 2
    for i in range(2):
        kvbuf[i * half : (i + 1) * half, :] = jnp.dot(
            _bf16(xbuf[i * half : (i + 1) * half, :]),
            _bf16(wkvbuf[...]),
            preferred_element_type=jnp.float32,
        )
        c = pltpu.make_async_copy(
            kvbuf.at[i * half : (i + 1) * half, :],
            kv_hbm.at[i * half : (i + 1) * half, :],
            sems.at[n],
        )
        c.start()
        copies.append(c)
        n += 1
    k = kvbuf[:, :KV_D]

    in_q.wait()
    q = jnp.dot(
        _bf16(xbuf[S - 1 : S, :]),
        _bf16(wqbuf[...]),
        preferred_element_type=jnp.float32,
    )
    rows = []
    for h in range(NUM_HEADS):
        qh = _bf16(q[:, h * HEAD_DIM : (h + 1) * HEAD_DIM])
        g = h // N_REP
        kg = _bf16(k[:, g * HEAD_DIM : (g + 1) * HEAD_DIM])
        rows.append(
            jax.lax.dot_general(
                qh, kg, (((1,), (1,)), ((), ())),
                preferred_element_type=jnp.float32,
            )
        )
    scores = jnp.concatenate(rows, axis=0) * (
        1.0 / np.sqrt(HEAD_DIM)
    )
    m = jnp.max(scores, axis=1, keepdims=True)
    e = jnp.exp(scores - m)
    attn = e / jnp.sum(e, axis=1, keepdims=True)
    attnbuf[...] = attn

    imp = jnp.mean(attn, axis=0, keepdims=True)
    lane = jax.lax.broadcasted_iota(jnp.int32, (1, S), 1)
    is_pid = functools.reduce(jnp.logical_or, [lane == p for p in _PID])
    impbuf[...] = jnp.where(is_pid, imp * 2.5, imp)

    c = pltpu.make_async_copy(attnbuf, attn_hbm, sems.at[n])
    c.start()
    copies.append(c)
    n += 1
    c = pltpu.make_async_copy(impbuf, imp_hbm, sems.at[n])
    c.start()
    copies.append(c)
    n += 1

    for c in copies:
        c.wait()


def _sc_topk_kernel(imp_hbm, sel_hbm, impv, bitsv, selv):
    cid = lax.axis_index("c")
    sid = lax.axis_index("s")

    @pl.when(jnp.logical_and(cid == 0, sid == 0))
    def _():
        pltpu.sync_copy(imp_hbm, impv)

        def tobits(i, carry):
            for u in range(4):
                v = impv[pl.ds((4 * i + u) * 16, 16)]
                bitsv[pl.ds((4 * i + u) * 16, 16)] = plsc.bitcast(
                    v, jnp.int32
                )
            return carry

        lax.fori_loop(0, _NV // 4, tobits, jnp.int32(0))

        def count_ge(cand):
            def inner(i, cnt):
                acc = jnp.zeros((16,), jnp.int32)
                for u in range(4):
                    bv = bitsv[pl.ds((4 * i + u) * 16, 16)]
                    acc = acc + jnp.where(
                        bv >= cand, jnp.int32(1), jnp.int32(0)
                    )
                return cnt + jnp.sum(acc)

            return lax.fori_loop(0, _NV // 4, inner, jnp.int32(0))

        def outer(b, t):
            cand = t | lax.shift_left(jnp.int32(1), 30 - b)
            return jnp.where(count_ge(cand) >= _KK, cand, t)

        t = lax.fori_loop(0, 31, outer, jnp.int32(0))

        ngt = count_ge(t + 1)
        need = _KK - ngt

        def selpass(i, run):
            bv = bitsv[pl.ds(i * 16, 16)]
            eqv = bv == t
            ones = jnp.where(eqv, jnp.int32(1), jnp.int32(0))
            cs = lax.cumsum(ones) + run
            keep = jnp.logical_or(
                bv > t, jnp.logical_and(eqv, cs <= need)
            )
            selv[pl.ds(i * 16, 16)] = jnp.where(keep, 1.0, 0.0)
            return run + jnp.sum(ones)

        lax.fori_loop(0, _NV, selpass, jnp.int32(0))
        pltpu.sync_copy(selv, sel_hbm)


def _sc_topk(imp):
    mesh = plsc.VectorSubcoreMesh(core_axis_name="c", subcore_axis_name="s")
    kfn = functools.partial(
        pl.kernel,
        mesh=mesh,
        out_type=jax.ShapeDtypeStruct((S,), jnp.float32),
        scratch_types=[
            pltpu.VMEM((S,), jnp.float32),
            pltpu.VMEM((S,), jnp.int32),
            pltpu.VMEM((S,), jnp.float32),
        ],
        compiler_params=pltpu.CompilerParams(needs_layout_passes=False),
    )(_sc_topk_kernel)
    return kfn(imp)


def _tail_kernel(
    attn_ref, sel_ref, kv_hbm, wo_hbm,
    pruned_in, out_in,
    pruned_ref, out_ref,
    vbuf, wobuf, prow_buf, olast_buf, sems,
):
    del pruned_in, out_in
    in_v = pltpu.make_async_copy(
        kv_hbm.at[:, KV_D : 2 * KV_D], vbuf, sems.at[30]
    )
    in_o = pltpu.make_async_copy(wo_hbm, wobuf, sems.at[31])
    in_v.start()
    in_o.start()

    prow = attn_ref[...] * sel_ref[...]
    prow_buf[...] = jnp.zeros_like(prow_buf)
    for h in range(NUM_HEADS):
        prow_buf[8 * h + 7 : 8 * h + 8, :] = prow[h : h + 1, :]

    copies = []
    n = 0
    for h in range(NUM_HEADS):
        c = pltpu.make_async_copy(
            prow_buf.at[8 * h : 8 * (h + 1), :],
            pruned_ref.at[h, S - 8 : S, :],
            sems.at[n],
        )
        c.start()
        copies.append(c)
        n += 1

    in_v.wait()
    ctx = jnp.dot(
        _bf16(prow), _bf16(vbuf[...]), preferred_element_type=jnp.float32
    )
    hh = jax.lax.broadcasted_iota(jnp.int32, (NUM_HEADS, KV_D), 0)
    gg = jax.lax.broadcasted_iota(jnp.int32, (NUM_HEADS, KV_D), 1) // HEAD_DIM
    ctx = jnp.where(hh // N_REP == gg, ctx, 0.0)
    ctx16 = (
        ctx[:, 0:64] + ctx[:, 64:128] + ctx[:, 128:192] + ctx[:, 192:256]
    )

    in_o.wait()
    olast = jnp.zeros((1, D_MODEL), jnp.float32)
    for h in range(NUM_HEADS):
        olast = olast + jnp.dot(
            _bf16(ctx16[h : h + 1, :]),
            _bf16(wobuf[h * HEAD_DIM : (h + 1) * HEAD_DIM, :]),
            preferred_element_type=jnp.float32,
        )
    olast_buf[...] = jnp.zeros_like(olast_buf)
    olast_buf[7:8, :] = olast

    c = pltpu.make_async_copy(olast_buf, out_ref.at[S - 8 : S, :], sems.at[n])
    c.start()
    copies.append(c)
    n += 1

    for c in copies:
        c.wait()


def kernel(hidden_states, Wq, Wk, Wv, Wo):
    x = hidden_states[0]
    Wkv = jnp.concatenate([Wk, Wv], axis=1)

    hbm = pl.BlockSpec(memory_space=pltpu.MemorySpace.HBM)
    vmem = pl.BlockSpec(memory_space=pltpu.MemorySpace.VMEM)
    pruned0, out0, kv, attn, imp = pl.pallas_call(
        _mega_kernel,
        in_specs=[hbm, hbm, hbm],
        out_specs=(hbm, hbm, hbm, hbm, hbm),
        out_shape=(
            jax.ShapeDtypeStruct((NUM_HEADS, S, S), jnp.float32),
            jax.ShapeDtypeStruct((S, D_MODEL), jnp.float32),
            jax.ShapeDtypeStruct((S, 2 * KV_D), jnp.float32),
            jax.ShapeDtypeStruct((NUM_HEADS, S), jnp.float32),
            jax.ShapeDtypeStruct((1, S), jnp.float32),
        ),
        scratch_shapes=[
            pltpu.VMEM((S, D_MODEL), jnp.float32),
            pltpu.VMEM((D_MODEL, D_MODEL), jnp.float32),
            pltpu.VMEM((D_MODEL, 2 * KV_D), jnp.float32),
            pltpu.VMEM((S, 2 * KV_D), jnp.float32),
            pltpu.VMEM((_ZROWS, S), jnp.float32),
            pltpu.VMEM((_ZROWS, S), jnp.float32),
            pltpu.VMEM((NUM_HEADS, S), jnp.float32),
            pltpu.VMEM((1, S), jnp.float32),
            pltpu.SemaphoreType.DMA((128,)),
        ],
    )(x, Wq, Wkv)

    sel = _sc_topk(imp.reshape(S))

    pruned, out = pl.pallas_call(
        _tail_kernel,
        in_specs=[vmem, vmem, hbm, hbm, hbm, hbm],
        out_specs=(hbm, hbm),
        out_shape=(
            jax.ShapeDtypeStruct((NUM_HEADS, S, S), jnp.float32),
            jax.ShapeDtypeStruct((S, D_MODEL), jnp.float32),
        ),
        input_output_aliases={4: 0, 5: 1},
        scratch_shapes=[
            pltpu.VMEM((S, KV_D), jnp.float32),
            pltpu.VMEM((D_MODEL, D_MODEL), jnp.float32),
            pltpu.VMEM((8 * NUM_HEADS, S), jnp.float32),
            pltpu.VMEM((8, D_MODEL), jnp.float32),
            pltpu.SemaphoreType.DMA((64,)),
        ],
    )(attn, sel.reshape(1, S), kv, Wo, pruned0, out0)

    k_flat = kv[:, :KV_D]
    v_flat = kv[:, KV_D:]
    k_kv = k_flat.reshape(1, S, NUM_KV_HEADS, HEAD_DIM).transpose(0, 2, 1, 3)
    v_kv = v_flat.reshape(1, S, NUM_KV_HEADS, HEAD_DIM).transpose(0, 2, 1, 3)
    return out[None], pruned[None], k_kv, v_kv


# speedup vs baseline: 1.0956x; 1.0163x over previous
import functools

import jax
import jax.numpy as jnp
import numpy as np
from jax import lax
from jax.experimental import pallas as pl
from jax.experimental.pallas import tpu as pltpu
from jax.experimental.pallas import tpu_sc as plsc

S = 2048
D_MODEL = 1024
NUM_HEADS = 16
NUM_KV_HEADS = 4
HEAD_DIM = 64
N_REP = NUM_HEADS // NUM_KV_HEADS
KV_D = NUM_KV_HEADS * HEAD_DIM
_PID = (0, 1, 2, 3, 50, 100)
_LAYER_IDX = 8
_KK = int(192 - _LAYER_IDX / 31 * (192 - 64))

_ZROWS = 512
_NV = S // 16


def _bf16(a):
    return a.astype(jnp.bfloat16)


def _mega_kernel(
    x_hbm, wq_hbm, wkv_hbm,
    pruned_ref, out_ref, kv_hbm, attn_hbm, imp_hbm,
    xbuf, wqbuf, wkvbuf, kvbuf, zbuf, zbuf2, attnbuf, impbuf, sems,
):
    in_x = pltpu.make_async_copy(x_hbm, xbuf, sems.at[120])
    in_q = pltpu.make_async_copy(wq_hbm, wqbuf, sems.at[121])
    in_kv = pltpu.make_async_copy(wkv_hbm, wkvbuf, sems.at[122])
    for c in (in_x, in_kv, in_q):
        c.start()

    zbuf[...] = jnp.zeros_like(zbuf)
    zbuf2[...] = jnp.zeros_like(zbuf2)
    zsrcs = (zbuf, zbuf2)
    copies = []
    n = 0
    nblk = (S - 8) // _ZROWS
    rem = (S - 8) - nblk * _ZROWS
    for h in range(NUM_HEADS):
        for j in range(nblk):
            zsrc = zsrcs[(h * (nblk + 1) + j) % 2]
            c = pltpu.make_async_copy(
                zsrc,
                pruned_ref.at[h, j * _ZROWS : (j + 1) * _ZROWS, :],
                sems.at[n],
            )
            c.start()
            copies.append(c)
            n += 1
        zsrc = zsrcs[(h * (nblk + 1) + nblk) % 2]
        c = pltpu.make_async_copy(
            zsrc.at[0:rem, :],
            pruned_ref.at[h, nblk * _ZROWS : S - 8, :],
            sems.at[n],
        )
        c.start()
        copies.append(c)
        n += 1
    for j in range(nblk):
        c = pltpu.make_async_copy(
            zsrcs[j % 2].at[:, 0:D_MODEL],
            out_ref.at[j * _ZROWS : (j + 1) * _ZROWS, :],
            sems.at[n],
        )
        c.start()
        copies.append(c)
        n += 1
    c = pltpu.make_async_copy(
        zbuf.at[0:rem, 0:D_MODEL],
        out_ref.at[nblk * _ZROWS : S - 8, :],
        sems.at[n],
    )
    c.start()
    copies.append(c)
    n += 1

    in_x.wait()
    in_kv.wait()
    half = S // 2
    for i in range(2):
        kvbuf[i * half : (i + 1) * half, :] = jnp.dot(
            _bf16(xbuf[i * half : (i + 1) * half, :]),
            _bf16(wkvbuf[...]),
            preferred_element_type=jnp.float32,
        )
        c = pltpu.make_async_copy(
            kvbuf.at[i * half : (i + 1) * half, :],
            kv_hbm.at[i * half : (i + 1) * half, :],
            sems.at[n],
        )
        c.start()
        copies.append(c)
        n += 1
    k = kvbuf[:, :KV_D]

    in_q.wait()
    q = jnp.dot(
        _bf16(xbuf[S - 1 : S, :]),
        _bf16(wqbuf[...]),
        preferred_element_type=jnp.float32,
    )
    rows = []
    for h in range(NUM_HEADS):
        qh = _bf16(q[:, h * HEAD_DIM : (h + 1) * HEAD_DIM])
        g = h // N_REP
        kg = _bf16(k[:, g * HEAD_DIM : (g + 1) * HEAD_DIM])
        rows.append(
            jax.lax.dot_general(
                qh, kg, (((1,), (1,)), ((), ())),
                preferred_element_type=jnp.float32,
            )
        )
    scores = jnp.concatenate(rows, axis=0) * (
        1.0 / np.sqrt(HEAD_DIM)
    )
    m = jnp.max(scores, axis=1, keepdims=True)
    e = jnp.exp(scores - m)
    attn = e / jnp.sum(e, axis=1, keepdims=True)
    attnbuf[...] = attn

    imp = jnp.mean(attn, axis=0, keepdims=True)
    lane = jax.lax.broadcasted_iota(jnp.int32, (1, S), 1)
    is_pid = functools.reduce(jnp.logical_or, [lane == p for p in _PID])
    impbuf[...] = jnp.where(is_pid, imp * 2.5, imp)

    c = pltpu.make_async_copy(attnbuf, attn_hbm, sems.at[n])
    c.start()
    copies.append(c)
    n += 1
    c = pltpu.make_async_copy(impbuf, imp_hbm, sems.at[n])
    c.start()
    copies.append(c)
    n += 1

    for c in copies:
        c.wait()


def _sc_topk_kernel(imp_hbm, sel_hbm, impv, bitsv, selv):
    cid = lax.axis_index("c")
    sid = lax.axis_index("s")

    @pl.when(jnp.logical_and(cid == 0, sid == 0))
    def _():
        pltpu.sync_copy(imp_hbm, impv)

        def tobits(i, carry):
            for u in range(4):
                v = impv[pl.ds((4 * i + u) * 16, 16)]
                bitsv[pl.ds((4 * i + u) * 16, 16)] = plsc.bitcast(
                    v, jnp.int32
                )
            return carry

        lax.fori_loop(0, _NV // 4, tobits, jnp.int32(0))

        def count_ge(cand):
            def inner(i, cnt):
                acc = jnp.zeros((16,), jnp.int32)
                for u in range(16):
                    bv = bitsv[pl.ds((16 * i + u) * 16, 16)]
                    acc = acc + jnp.where(
                        bv >= cand, jnp.int32(1), jnp.int32(0)
                    )
                return cnt + jnp.sum(acc)

            return lax.fori_loop(0, _NV // 16, inner, jnp.int32(0))

        def outer(b, t):
            cand = t | lax.shift_left(jnp.int32(1), 30 - b)
            return jnp.where(count_ge(cand) >= _KK, cand, t)

        t = lax.fori_loop(0, 31, outer, jnp.int32(0))

        ngt = count_ge(t + 1)
        need = _KK - ngt

        def selpass(i, run):
            bv = bitsv[pl.ds(i * 16, 16)]
            eqv = bv == t
            ones = jnp.where(eqv, jnp.int32(1), jnp.int32(0))
            cs = lax.cumsum(ones) + run
            keep = jnp.logical_or(
                bv > t, jnp.logical_and(eqv, cs <= need)
            )
            selv[pl.ds(i * 16, 16)] = jnp.where(keep, 1.0, 0.0)
            return run + jnp.sum(ones)

        lax.fori_loop(0, _NV, selpass, jnp.int32(0))
        pltpu.sync_copy(selv, sel_hbm)


def _sc_topk(imp):
    mesh = plsc.VectorSubcoreMesh(core_axis_name="c", subcore_axis_name="s")
    kfn = functools.partial(
        pl.kernel,
        mesh=mesh,
        out_type=jax.ShapeDtypeStruct((S,), jnp.float32),
        scratch_types=[
            pltpu.VMEM((S,), jnp.float32),
            pltpu.VMEM((S,), jnp.int32),
            pltpu.VMEM((S,), jnp.float32),
        ],
        compiler_params=pltpu.CompilerParams(needs_layout_passes=False),
    )(_sc_topk_kernel)
    return kfn(imp)


def _tail_kernel(
    attn_ref, sel_ref, kv_hbm, wo_hbm,
    pruned_in, out_in,
    pruned_ref, out_ref,
    vbuf, wobuf, prow_buf, olast_buf, sems,
):
    del pruned_in, out_in
    in_v = pltpu.make_async_copy(
        kv_hbm.at[:, KV_D : 2 * KV_D], vbuf, sems.at[30]
    )
    in_o = pltpu.make_async_copy(wo_hbm, wobuf, sems.at[31])
    in_v.start()
    in_o.start()

    prow = attn_ref[...] * sel_ref[...]
    prow_buf[...] = jnp.zeros_like(prow_buf)
    for h in range(NUM_HEADS):
        prow_buf[8 * h + 7 : 8 * h + 8, :] = prow[h : h + 1, :]

    copies = []
    n = 0
    for h in range(NUM_HEADS):
        c = pltpu.make_async_copy(
            prow_buf.at[8 * h : 8 * (h + 1), :],
            pruned_ref.at[h, S - 8 : S, :],
            sems.at[n],
        )
        c.start()
        copies.append(c)
        n += 1

    in_v.wait()
    ctx = jnp.dot(
        _bf16(prow), _bf16(vbuf[...]), preferred_element_type=jnp.float32
    )
    hh = jax.lax.broadcasted_iota(jnp.int32, (NUM_HEADS, KV_D), 0)
    gg = jax.lax.broadcasted_iota(jnp.int32, (NUM_HEADS, KV_D), 1) // HEAD_DIM
    ctx = jnp.where(hh // N_REP == gg, ctx, 0.0)
    ctx16 = (
        ctx[:, 0:64] + ctx[:, 64:128] + ctx[:, 128:192] + ctx[:, 192:256]
    )

    in_o.wait()
    olast = jnp.zeros((1, D_MODEL), jnp.float32)
    for h in range(NUM_HEADS):
        olast = olast + jnp.dot(
            _bf16(ctx16[h : h + 1, :]),
            _bf16(wobuf[h * HEAD_DIM : (h + 1) * HEAD_DIM, :]),
            preferred_element_type=jnp.float32,
        )
    olast_buf[...] = jnp.zeros_like(olast_buf)
    olast_buf[7:8, :] = olast

    c = pltpu.make_async_copy(olast_buf, out_ref.at[S - 8 : S, :], sems.at[n])
    c.start()
    copies.append(c)
    n += 1

    for c in copies:
        c.wait()


def kernel(hidden_states, Wq, Wk, Wv, Wo):
    x = hidden_states[0]
    Wkv = jnp.concatenate([Wk, Wv], axis=1)

    hbm = pl.BlockSpec(memory_space=pltpu.MemorySpace.HBM)
    vmem = pl.BlockSpec(memory_space=pltpu.MemorySpace.VMEM)
    pruned0, out0, kv, attn, imp = pl.pallas_call(
        _mega_kernel,
        in_specs=[hbm, hbm, hbm],
        out_specs=(hbm, hbm, hbm, hbm, hbm),
        out_shape=(
            jax.ShapeDtypeStruct((NUM_HEADS, S, S), jnp.float32),
            jax.ShapeDtypeStruct((S, D_MODEL), jnp.float32),
            jax.ShapeDtypeStruct((S, 2 * KV_D), jnp.float32),
            jax.ShapeDtypeStruct((NUM_HEADS, S), jnp.float32),
            jax.ShapeDtypeStruct((1, S), jnp.float32),
        ),
        scratch_shapes=[
            pltpu.VMEM((S, D_MODEL), jnp.float32),
            pltpu.VMEM((D_MODEL, D_MODEL), jnp.float32),
            pltpu.VMEM((D_MODEL, 2 * KV_D), jnp.float32),
            pltpu.VMEM((S, 2 * KV_D), jnp.float32),
            pltpu.VMEM((_ZROWS, S), jnp.float32),
            pltpu.VMEM((_ZROWS, S), jnp.float32),
            pltpu.VMEM((NUM_HEADS, S), jnp.float32),
            pltpu.VMEM((1, S), jnp.float32),
            pltpu.SemaphoreType.DMA((128,)),
        ],
    )(x, Wq, Wkv)

    sel = _sc_topk(imp.reshape(S))

    pruned, out = pl.pallas_call(
        _tail_kernel,
        in_specs=[vmem, vmem, hbm, hbm, hbm, hbm],
        out_specs=(hbm, hbm),
        out_shape=(
            jax.ShapeDtypeStruct((NUM_HEADS, S, S), jnp.float32),
            jax.ShapeDtypeStruct((S, D_MODEL), jnp.float32),
        ),
        input_output_aliases={4: 0, 5: 1},
        scratch_shapes=[
            pltpu.VMEM((S, KV_D), jnp.float32),
            pltpu.VMEM((D_MODEL, D_MODEL), jnp.float32),
            pltpu.VMEM((8 * NUM_HEADS, S), jnp.float32),
            pltpu.VMEM((8, D_MODEL), jnp.float32),
            pltpu.SemaphoreType.DMA((64,)),
        ],
    )(attn, sel.reshape(1, S), kv, Wo, pruned0, out0)

    k_flat = kv[:, :KV_D]
    v_flat = kv[:, KV_D:]
    k_kv = k_flat.reshape(1, S, NUM_KV_HEADS, HEAD_DIM).transpose(0, 2, 1, 3)
    v_kv = v_flat.reshape(1, S, NUM_KV_HEADS, HEAD_DIM).transpose(0, 2, 1, 3)
    return out[None], pruned[None], k_kv, v_kv
